# baseline (speedup 1.0000x reference)
import functools

import jax
import jax.numpy as jnp
from jax import lax
from jax.experimental import pallas as pl
from jax.experimental.pallas import tpu as pltpu
from jax.experimental.pallas import tpu_sc as plsc

B = 16384
D = 64
CHUNK = 128

_info = plsc.get_sparse_core_info()
NC, NS = _info.num_cores, _info.num_subcores
NW = NC * NS
BPW = B // NW
NCH = BPW // CHUNK


def _sc_gather(nodes3, audio_tab, aid_tab, bid_tab, artist_tab, album_tab):
    mesh = plsc.VectorSubcoreMesh(core_axis_name="c", subcore_axis_name="s")

    @functools.partial(
        pl.kernel,
        mesh=mesh,
        compiler_params=pltpu.CompilerParams(use_tc_tiling_on_sc=False),
        out_type=(
            jax.ShapeDtypeStruct((B, D), jnp.float32),
            jax.ShapeDtypeStruct((B, D), jnp.float32),
            jax.ShapeDtypeStruct((B, D), jnp.float32),
        ),
        scratch_types=[
            pltpu.VMEM((NCH, CHUNK), jnp.int32),
            pltpu.VMEM((NCH, CHUNK), jnp.int32),
            pltpu.VMEM((NCH, CHUNK), jnp.int32),
            pltpu.VMEM((BPW, D), jnp.float32),
            pltpu.VMEM((BPW, D), jnp.float32),
            pltpu.VMEM((BPW, D), jnp.float32),
            pltpu.SemaphoreType.DMA,
            pltpu.SemaphoreType.DMA,
        ],
    )
    def k(nodes_hbm, audio_hbm, aid_hbm, bid_hbm, atab_hbm, btab_hbm,
          audio_out, art_out, alb_out,
          idx_v, aid_v, bid_v, audio_v, art_v, alb_v, sem_ids, sem_rows):
        wid = lax.axis_index("s") * NC + lax.axis_index("c")
        base = wid * BPW
        pltpu.sync_copy(nodes_hbm.at[wid], idx_v)
        for j in range(NCH):
            pltpu.async_copy(
                audio_hbm.at[idx_v.at[j]],
                audio_v.at[pl.ds(j * CHUNK, CHUNK)], sem_rows).wait()
            pltpu.async_copy(
                aid_hbm.at[idx_v.at[j]], aid_v.at[j], sem_ids).wait()
            pltpu.async_copy(
                bid_hbm.at[idx_v.at[j]], bid_v.at[j], sem_ids).wait()
        for j in range(NCH):
            pltpu.async_copy(
                atab_hbm.at[aid_v.at[j]],
                art_v.at[pl.ds(j * CHUNK, CHUNK)], sem_rows).wait()
            pltpu.async_copy(
                btab_hbm.at[bid_v.at[j]],
                alb_v.at[pl.ds(j * CHUNK, CHUNK)], sem_rows).wait()
        pltpu.sync_copy(audio_v, audio_out.at[pl.ds(base, BPW)])
        pltpu.sync_copy(art_v, art_out.at[pl.ds(base, BPW)])
        pltpu.sync_copy(alb_v, alb_out.at[pl.ds(base, BPW)])

    return k(nodes3, audio_tab, aid_tab, bid_tab, artist_tab, album_tab)


BLK = 2048


def _tc_project(audio, art, alb, W, b2):
    def body(a_ref, r_ref, l_ref, w_ref, b_ref, o_ref):
        a = a_ref[...]
        m = r_ref[...] + l_ref[...]
        w = w_ref[...]
        y = (jnp.dot(a, w[:D], preferred_element_type=jnp.float32,
                     precision=lax.Precision.HIGHEST)
             + jnp.dot(m, w[D:], preferred_element_type=jnp.float32,
                       precision=lax.Precision.HIGHEST)
             + b_ref[...])
        s = jnp.sum(y * y, axis=-1, keepdims=True)
        n = jnp.sqrt(s)
        o_ref[...] = y / jnp.maximum(n, 1e-12)

    return pl.pallas_call(
        body,
        grid=(B // BLK,),
        in_specs=[
            pl.BlockSpec((BLK, D), lambda i: (i, 0)),
            pl.BlockSpec((BLK, D), lambda i: (i, 0)),
            pl.BlockSpec((BLK, D), lambda i: (i, 0)),
            pl.BlockSpec((2 * D, D), lambda i: (0, 0)),
            pl.BlockSpec((1, D), lambda i: (0, 0)),
        ],
        out_specs=pl.BlockSpec((BLK, D), lambda i: (i, 0)),
        out_shape=jax.ShapeDtypeStruct((B, D), jnp.float32),
    )(audio, art, alb, W, b2)


def kernel(item_nodes, item_audio_emb, artist_ids, album_ids,
           artist_table, album_table, W, b):
    nodes3 = item_nodes.astype(jnp.int32).reshape(NW, NCH, CHUNK)
    audio, art, alb = _sc_gather(
        nodes3, item_audio_emb,
        artist_ids.astype(jnp.int32), album_ids.astype(jnp.int32),
        artist_table, album_table)
    return _tc_project(audio, art, alb, W, b.reshape(1, D))


# speedup vs baseline: 1.0163x; 1.0163x over previous
import functools

import jax
import jax.numpy as jnp
from jax import lax
from jax.experimental import pallas as pl
from jax.experimental.pallas import tpu as pltpu
from jax.experimental.pallas import tpu_sc as plsc

B = 16384
D = 64
CHUNK = 128

_info = plsc.get_sparse_core_info()
NC, NS = _info.num_cores, _info.num_subcores
NW = NC * NS
BPW = B // NW
NCH = BPW // CHUNK


def _sc_gather(nodes3, audio_tab, aid_tab, bid_tab, artist_tab, album_tab):
    mesh = plsc.VectorSubcoreMesh(core_axis_name="c", subcore_axis_name="s")

    @functools.partial(
        pl.kernel,
        mesh=mesh,
        compiler_params=pltpu.CompilerParams(use_tc_tiling_on_sc=False),
        out_type=(
            jax.ShapeDtypeStruct((B, D), jnp.float32),
            jax.ShapeDtypeStruct((B, D), jnp.float32),
            jax.ShapeDtypeStruct((B, D), jnp.float32),
        ),
        scratch_types=[
            pltpu.VMEM((BPW,), jnp.int32),
            pltpu.VMEM((BPW,), jnp.int32),
            pltpu.VMEM((BPW,), jnp.int32),
            pltpu.VMEM((BPW, D), jnp.float32),
            pltpu.VMEM((BPW, D), jnp.float32),
            pltpu.VMEM((BPW, D), jnp.float32),
            pltpu.SemaphoreType.DMA,
            pltpu.SemaphoreType.DMA,
            pltpu.SemaphoreType.DMA,
        ],
    )
    def k(nodes_hbm, audio_hbm, aid_hbm, bid_hbm, atab_hbm, btab_hbm,
          audio_out, art_out, alb_out,
          idx_v, aid_v, bid_v, audio_v, art_v, alb_v,
          sem_ids, sem_audio, sem_tab):
        wid = lax.axis_index("s") * NC + lax.axis_index("c")
        base = wid * BPW
        pltpu.sync_copy(nodes_hbm.at[wid], idx_v)
        c_audio = pltpu.async_copy(audio_hbm.at[idx_v], audio_v, sem_audio)
        c_aid = pltpu.async_copy(aid_hbm.at[idx_v], aid_v, sem_ids)
        c_bid = pltpu.async_copy(bid_hbm.at[idx_v], bid_v, sem_ids)
        c_aid.wait()
        c_bid.wait()
        c_art = pltpu.async_copy(atab_hbm.at[aid_v], art_v, sem_tab)
        c_alb = pltpu.async_copy(btab_hbm.at[bid_v], alb_v, sem_tab)
        c_audio.wait()
        pltpu.sync_copy(audio_v, audio_out.at[pl.ds(base, BPW)])
        c_art.wait()
        pltpu.sync_copy(art_v, art_out.at[pl.ds(base, BPW)])
        c_alb.wait()
        pltpu.sync_copy(alb_v, alb_out.at[pl.ds(base, BPW)])

    return k(nodes3, audio_tab, aid_tab, bid_tab, artist_tab, album_tab)


BLK = 2048


def _tc_project(audio, art, alb, W, b2):
    def body(a_ref, r_ref, l_ref, w_ref, b_ref, o_ref):
        a = a_ref[...]
        m = r_ref[...] + l_ref[...]
        w = w_ref[...]
        y = (jnp.dot(a, w[:D], preferred_element_type=jnp.float32,
                     precision=lax.Precision.HIGHEST)
             + jnp.dot(m, w[D:], preferred_element_type=jnp.float32,
                       precision=lax.Precision.HIGHEST)
             + b_ref[...])
        s = jnp.sum(y * y, axis=-1, keepdims=True)
        n = jnp.sqrt(s)
        o_ref[...] = y / jnp.maximum(n, 1e-12)

    return pl.pallas_call(
        body,
        grid=(B // BLK,),
        in_specs=[
            pl.BlockSpec((BLK, D), lambda i: (i, 0)),
            pl.BlockSpec((BLK, D), lambda i: (i, 0)),
            pl.BlockSpec((BLK, D), lambda i: (i, 0)),
            pl.BlockSpec((2 * D, D), lambda i: (0, 0)),
            pl.BlockSpec((1, D), lambda i: (0, 0)),
        ],
        out_specs=pl.BlockSpec((BLK, D), lambda i: (i, 0)),
        out_shape=jax.ShapeDtypeStruct((B, D), jnp.float32),
    )(audio, art, alb, W, b2)


def kernel(item_nodes, item_audio_emb, artist_ids, album_ids,
           artist_table, album_table, W, b):
    nodes3 = item_nodes.astype(jnp.int32).reshape(NW, BPW)
    audio, art, alb = _sc_gather(
        nodes3, item_audio_emb,
        artist_ids.astype(jnp.int32), album_ids.astype(jnp.int32),
        artist_table, album_table)
    return _tc_project(audio, art, alb, W, b.reshape(1, D))
